# baseline (speedup 1.0000x reference)
import functools

import jax
import jax.numpy as jnp
from jax import lax
from jax.experimental import pallas as pl
from jax.experimental.pallas import tpu as pltpu
from jax.experimental.pallas import tpu_sc as plsc

T = 16384
E = 8
D = 4096
TOPK = 2
FLAT = T * E
OUT = T * TOPK
NW = 32
CHUNK = FLAT // NW
ROWS_PW = OUT // NW
L = 16
RG = 8
NBUF = 2

_MESH = dict(core_axis_name="c", subcore_axis_name="s")

_DNUMS = lax.GatherDimensionNumbers(
    offset_dims=(), collapsed_slice_dims=(0,), start_index_map=(0,))


def _wid():
    return lax.axis_index("s") * 2 + lax.axis_index("c")


def _dyn_gather(x, idx):
    return lax.gather(x, idx[:, None], dimension_numbers=_DNUMS,
                      slice_sizes=(1,),
                      mode=lax.GatherScatterMode.PROMISE_IN_BOUNDS)


def _iota():
    return lax.iota(jnp.int32, L)


def _shift_cumsum(x):
    iota = _iota()
    y = x
    for s in (1, 2, 4, 8):
        sh = _dyn_gather(y, jnp.maximum(iota - s, 0))
        y = y + jnp.where(iota >= s, sh, 0)
    return y


def _sum_scalar(v):
    return _shift_cumsum(v)[L - 1]


def _splat(x):
    return jnp.full((L,), x, jnp.int32)


@functools.partial(
    pl.kernel,
    out_type=(
        jax.ShapeDtypeStruct((NW * 8,), jnp.int32),
        jax.ShapeDtypeStruct((FLAT,), jnp.int32),
        jax.ShapeDtypeStruct((FLAT,), jnp.float32),
    ),
    mesh=plsc.VectorSubcoreMesh(**_MESH),
    compiler_params=pltpu.CompilerParams(needs_layout_passes=False),
    scratch_types=[
        pltpu.VMEM((CHUNK,), jnp.int32),
        pltpu.VMEM((CHUNK,), jnp.float32),
        pltpu.VMEM((CHUNK,), jnp.int32),
        pltpu.VMEM((CHUNK,), jnp.float32),
        pltpu.VMEM((L,), jnp.int32),
    ],
)
def _compact_kernel(maskf_hbm, probsf_hbm, counts_hbm, selpad_hbm, prpad_hbm,
                    chunk_v, pchunk_v, sel_buf, pr_buf, cs_v):
    w = _wid()
    iota = _iota()

    pltpu.sync_copy(maskf_hbm.at[pl.ds(w * CHUNK, CHUNK)], chunk_v)
    pltpu.sync_copy(probsf_hbm.at[pl.ds(w * CHUNK, CHUNK)], pchunk_v)
    base_flat = w * CHUNK

    def comp(g, off_vec):
        m = chunk_v[pl.ds(g * L, L)]
        mask = m == 1
        incl = _shift_cumsum(m)
        dst = jnp.maximum(off_vec + incl - 1, 0)
        plsc.store_scatter(sel_buf, [dst], base_flat + g * L + iota,
                           mask=mask)
        plsc.store_scatter(pr_buf, [dst], pchunk_v[pl.ds(g * L, L)],
                           mask=mask)
        return off_vec + plsc.all_reduce_population_count(mask)

    off_vec = lax.fori_loop(0, CHUNK // L, comp, jnp.zeros((L,), jnp.int32),
                            unroll=4)
    cs_v[...] = off_vec
    pltpu.sync_copy(cs_v.at[pl.ds(0, 8)], counts_hbm.at[pl.ds(w * 8, 8)])
    pltpu.sync_copy(sel_buf, selpad_hbm.at[pl.ds(base_flat, CHUNK)])
    pltpu.sync_copy(pr_buf, prpad_hbm.at[pl.ds(base_flat, CHUNK)])


@functools.partial(
    pl.kernel,
    out_type=(
        jax.ShapeDtypeStruct((OUT, D), jnp.float32),
        jax.ShapeDtypeStruct((E,), jnp.int32),
        jax.ShapeDtypeStruct((OUT,), jnp.float32),
    ),
    mesh=plsc.VectorSubcoreMesh(**_MESH),
    compiler_params=pltpu.CompilerParams(needs_layout_passes=False),
    scratch_types=[
        pltpu.VMEM((NW * 8,), jnp.int32),
        pltpu.VMEM((NW,), jnp.int32),
        pltpu.VMEM((NW,), jnp.int32),
        pltpu.VMEM((CHUNK,), jnp.int32),
        pltpu.VMEM((CHUNK,), jnp.float32),
        pltpu.VMEM((ROWS_PW,), jnp.int32),
        pltpu.VMEM((ROWS_PW,), jnp.int32),
        pltpu.VMEM((ROWS_PW,), jnp.float32),
        pltpu.VMEM((L,), jnp.int32),
        [pltpu.VMEM((RG, D), jnp.float32) for _ in range(NBUF)],
        [pltpu.SemaphoreType.DMA for _ in range(NBUF)],
        [pltpu.SemaphoreType.DMA for _ in range(NBUF)],
    ],
)
def _gather_kernel(hidden_hbm, counts_hbm, selpad_hbm, prpad_hbm,
                   out_hbm, tpe_hbm, pprobs_hbm,
                   cnt_v, pfx_v, ca_v, srow_v, prow_v,
                   sel_v, tok_v, pr_v, tpe_v, rbufs, isems, osems):
    w = _wid()
    iota = _iota()
    row0 = w * ROWS_PW

    pltpu.sync_copy(counts_hbm, cnt_v)
    c0 = plsc.load_gather(cnt_v, [iota * 8])
    c1 = plsc.load_gather(cnt_v, [(iota + L) * 8])
    p0 = _shift_cumsum(c0) - c0
    s0 = _sum_scalar(c0)
    p1 = _shift_cumsum(c1) - c1 + s0
    pfx_v[pl.ds(0, L)] = p0
    pfx_v[pl.ds(L, L)] = p1
    ca_v[pl.ds(0, L)] = c0
    ca_v[pl.ds(L, L)] = c1

    @pl.when(w == 0)
    def _():
        acc = jnp.zeros((L,), jnp.int32)
        for e in range(E):
            src = c0 if e < 4 else c1
            s = _sum_scalar(jnp.where(iota // 4 == (e % 4), src, 0))
            acc = acc + s * jnp.where(iota == e, 1, 0)
        tpe_v[...] = acc
        pltpu.sync_copy(tpe_v.at[pl.ds(0, E)], tpe_hbm)

    def pull(u, _):
        pu = plsc.load_gather(pfx_v, [_splat(u)])[0]
        cu = plsc.load_gather(ca_v, [_splat(u)])[0]
        a = jnp.maximum(pu, row0)
        b = jnp.minimum(pu + cu, row0 + ROWS_PW)
        n = b - a

        @pl.when(n > 0)
        def _():
            pltpu.sync_copy(selpad_hbm.at[pl.ds(u * CHUNK, CHUNK)], srow_v)
            pltpu.sync_copy(prpad_hbm.at[pl.ds(u * CHUNK, CHUNK)], prow_v)

            def cp(g, _):
                off = g * L + iota
                valid = off < n
                src = jnp.minimum(a - pu + off, CHUNK - 1)
                dst = jnp.minimum(a - row0 + off, ROWS_PW - 1)
                plsc.store_scatter(sel_v, [dst],
                                   plsc.load_gather(srow_v, [src]),
                                   mask=valid)
                plsc.store_scatter(pr_v, [dst],
                                   plsc.load_gather(prow_v, [src]),
                                   mask=valid)
                return 0

            lax.fori_loop(0, (n + L - 1) // L, cp, 0)

        return 0

    lax.fori_loop(0, NW, pull, 0)

    def tok(i, _):
        tok_v[pl.ds(i * L, L)] = lax.bitwise_and(sel_v[pl.ds(i * L, L)], T - 1)
        return 0

    lax.fori_loop(0, ROWS_PW // L, tok, 0, unroll=8)
    pltpu.sync_copy(pr_v, pprobs_hbm.at[pl.ds(row0, ROWS_PW)])

    ng = ROWS_PW // RG

    def start_in(g, b):
        pltpu.async_copy(hidden_hbm.at[tok_v.at[pl.ds(g * RG, RG)]],
                         rbufs[b], isems[b])

    def wait_in(b):
        pltpu.make_async_copy(hidden_hbm.at[tok_v.at[pl.ds(0, RG)]],
                              rbufs[b], isems[b]).wait()

    def start_out(g, b):
        pltpu.async_copy(rbufs[b], out_hbm.at[pl.ds(row0 + g * RG, RG)],
                         osems[b])

    def wait_out(b):
        pltpu.make_async_copy(rbufs[b], out_hbm.at[pl.ds(row0, RG)],
                              osems[b]).wait()

    for b in range(NBUF):
        start_in(b, b)

    def rows(i, _):
        for b in range(NBUF):
            wait_in(b)
            start_out(i * NBUF + b, b)
        for b in range(NBUF):
            g = i * NBUF + b

            @pl.when(g + NBUF < ng)
            def _():
                wait_out(b)
                start_in(g + NBUF, b)

        return 0

    lax.fori_loop(0, ng // NBUF, rows, 0)
    for b in range(NBUF):
        wait_out(b)


def kernel(hidden_states, probs, routing_map):
    maskf = routing_map.T.astype(jnp.int32).reshape(-1)
    probsf = probs.T.reshape(-1)
    counts, selpad, prpad = _compact_kernel(maskf, probsf)
    permuted_tokens, tokens_per_expert, permuted_probs = _gather_kernel(
        hidden_states, counts, selpad, prpad)
    return permuted_tokens, tokens_per_expert, permuted_probs


# speedup vs baseline: 1.0119x; 1.0119x over previous
import functools

import jax
import jax.numpy as jnp
from jax import lax
from jax.experimental import pallas as pl
from jax.experimental.pallas import tpu as pltpu
from jax.experimental.pallas import tpu_sc as plsc

T = 16384
E = 8
D = 4096
TOPK = 2
FLAT = T * E
OUT = T * TOPK
NW = 32
CHUNK = FLAT // NW
ROWS_PW = OUT // NW
L = 16
RG = 8
NBUF = 3

_MESH = dict(core_axis_name="c", subcore_axis_name="s")

_DNUMS = lax.GatherDimensionNumbers(
    offset_dims=(), collapsed_slice_dims=(0,), start_index_map=(0,))


def _wid():
    return lax.axis_index("s") * 2 + lax.axis_index("c")


def _dyn_gather(x, idx):
    return lax.gather(x, idx[:, None], dimension_numbers=_DNUMS,
                      slice_sizes=(1,),
                      mode=lax.GatherScatterMode.PROMISE_IN_BOUNDS)


def _iota():
    return lax.iota(jnp.int32, L)


def _shift_cumsum(x):
    iota = _iota()
    y = x
    for s in (1, 2, 4, 8):
        sh = _dyn_gather(y, jnp.maximum(iota - s, 0))
        y = y + jnp.where(iota >= s, sh, 0)
    return y


def _sum_scalar(v):
    return _shift_cumsum(v)[L - 1]


def _splat(x):
    return jnp.full((L,), x, jnp.int32)


@functools.partial(
    pl.kernel,
    out_type=(
        jax.ShapeDtypeStruct((NW * 8,), jnp.int32),
        jax.ShapeDtypeStruct((FLAT,), jnp.int32),
        jax.ShapeDtypeStruct((FLAT,), jnp.float32),
    ),
    mesh=plsc.VectorSubcoreMesh(**_MESH),
    compiler_params=pltpu.CompilerParams(needs_layout_passes=False),
    scratch_types=[
        pltpu.VMEM((CHUNK,), jnp.int32),
        pltpu.VMEM((CHUNK,), jnp.float32),
        pltpu.VMEM((CHUNK,), jnp.int32),
        pltpu.VMEM((CHUNK,), jnp.float32),
        pltpu.VMEM((L,), jnp.int32),
    ],
)
def _compact_kernel(maskf_hbm, probsf_hbm, counts_hbm, selpad_hbm, prpad_hbm,
                    chunk_v, pchunk_v, sel_buf, pr_buf, cs_v):
    w = _wid()
    iota = _iota()

    pltpu.sync_copy(maskf_hbm.at[pl.ds(w * CHUNK, CHUNK)], chunk_v)
    pltpu.sync_copy(probsf_hbm.at[pl.ds(w * CHUNK, CHUNK)], pchunk_v)
    base_flat = w * CHUNK

    def comp(g, off_vec):
        m = chunk_v[pl.ds(g * L, L)]
        mask = m == 1
        incl = _shift_cumsum(m)
        dst = jnp.maximum(off_vec + incl - 1, 0)
        plsc.store_scatter(sel_buf, [dst], base_flat + g * L + iota,
                           mask=mask)
        plsc.store_scatter(pr_buf, [dst], pchunk_v[pl.ds(g * L, L)],
                           mask=mask)
        return off_vec + plsc.all_reduce_population_count(mask)

    off_vec = lax.fori_loop(0, CHUNK // L, comp, jnp.zeros((L,), jnp.int32),
                            unroll=4)
    cs_v[...] = off_vec
    pltpu.sync_copy(cs_v.at[pl.ds(0, 8)], counts_hbm.at[pl.ds(w * 8, 8)])
    pltpu.sync_copy(sel_buf, selpad_hbm.at[pl.ds(base_flat, CHUNK)])
    pltpu.sync_copy(pr_buf, prpad_hbm.at[pl.ds(base_flat, CHUNK)])


@functools.partial(
    pl.kernel,
    out_type=(
        jax.ShapeDtypeStruct((OUT, D), jnp.float32),
        jax.ShapeDtypeStruct((E,), jnp.int32),
        jax.ShapeDtypeStruct((OUT,), jnp.float32),
    ),
    mesh=plsc.VectorSubcoreMesh(**_MESH),
    compiler_params=pltpu.CompilerParams(needs_layout_passes=False),
    scratch_types=[
        pltpu.VMEM((NW * 8,), jnp.int32),
        pltpu.VMEM((NW,), jnp.int32),
        pltpu.VMEM((NW,), jnp.int32),
        pltpu.VMEM((CHUNK,), jnp.int32),
        pltpu.VMEM((CHUNK,), jnp.float32),
        pltpu.VMEM((ROWS_PW,), jnp.int32),
        pltpu.VMEM((ROWS_PW,), jnp.int32),
        pltpu.VMEM((ROWS_PW,), jnp.float32),
        pltpu.VMEM((L,), jnp.int32),
        [pltpu.VMEM((RG, D), jnp.float32) for _ in range(NBUF)],
        [pltpu.SemaphoreType.DMA for _ in range(NBUF)],
        [pltpu.SemaphoreType.DMA for _ in range(NBUF)],
    ],
)
def _gather_kernel(hidden_hbm, counts_hbm, selpad_hbm, prpad_hbm,
                   out_hbm, tpe_hbm, pprobs_hbm,
                   cnt_v, pfx_v, ca_v, srow_v, prow_v,
                   sel_v, tok_v, pr_v, tpe_v, rbufs, isems, osems):
    w = _wid()
    iota = _iota()
    row0 = w * ROWS_PW

    pltpu.sync_copy(counts_hbm, cnt_v)
    c0 = plsc.load_gather(cnt_v, [iota * 8])
    c1 = plsc.load_gather(cnt_v, [(iota + L) * 8])
    p0 = _shift_cumsum(c0) - c0
    s0 = _sum_scalar(c0)
    p1 = _shift_cumsum(c1) - c1 + s0
    pfx_v[pl.ds(0, L)] = p0
    pfx_v[pl.ds(L, L)] = p1
    ca_v[pl.ds(0, L)] = c0
    ca_v[pl.ds(L, L)] = c1

    @pl.when(w == 0)
    def _():
        acc = jnp.zeros((L,), jnp.int32)
        for e in range(E):
            src = c0 if e < 4 else c1
            s = _sum_scalar(jnp.where(iota // 4 == (e % 4), src, 0))
            acc = acc + s * jnp.where(iota == e, 1, 0)
        tpe_v[...] = acc
        pltpu.sync_copy(tpe_v.at[pl.ds(0, E)], tpe_hbm)

    def pull(u, _):
        pu = plsc.load_gather(pfx_v, [_splat(u)])[0]
        cu = plsc.load_gather(ca_v, [_splat(u)])[0]
        a = jnp.maximum(pu, row0)
        b = jnp.minimum(pu + cu, row0 + ROWS_PW)
        n = b - a

        @pl.when(n > 0)
        def _():
            pltpu.sync_copy(selpad_hbm.at[pl.ds(u * CHUNK, CHUNK)], srow_v)
            pltpu.sync_copy(prpad_hbm.at[pl.ds(u * CHUNK, CHUNK)], prow_v)

            def cp(g, _):
                off = g * L + iota
                valid = off < n
                src = jnp.minimum(a - pu + off, CHUNK - 1)
                dst = jnp.minimum(a - row0 + off, ROWS_PW - 1)
                plsc.store_scatter(sel_v, [dst],
                                   plsc.load_gather(srow_v, [src]),
                                   mask=valid)
                plsc.store_scatter(pr_v, [dst],
                                   plsc.load_gather(prow_v, [src]),
                                   mask=valid)
                return 0

            lax.fori_loop(0, (n + L - 1) // L, cp, 0)

        return 0

    lax.fori_loop(0, NW, pull, 0)

    def tok(i, _):
        tok_v[pl.ds(i * L, L)] = lax.bitwise_and(sel_v[pl.ds(i * L, L)], T - 1)
        return 0

    lax.fori_loop(0, ROWS_PW // L, tok, 0, unroll=8)
    pltpu.sync_copy(pr_v, pprobs_hbm.at[pl.ds(row0, ROWS_PW)])

    ng = ROWS_PW // RG

    def start_in(g, b):
        pltpu.async_copy(hidden_hbm.at[tok_v.at[pl.ds(g * RG, RG)]],
                         rbufs[b], isems[b])

    def wait_in(b):
        pltpu.make_async_copy(hidden_hbm.at[tok_v.at[pl.ds(0, RG)]],
                              rbufs[b], isems[b]).wait()

    def start_out(g, b):
        pltpu.async_copy(rbufs[b], out_hbm.at[pl.ds(row0 + g * RG, RG)],
                         osems[b])

    def wait_out(b):
        pltpu.make_async_copy(rbufs[b], out_hbm.at[pl.ds(row0, RG)],
                              osems[b]).wait()

    for b in range(NBUF):
        start_in(b, b)

    def rows(i, _):
        for b in range(NBUF):
            wait_in(b)
            start_out(i * NBUF + b, b)
        for b in range(NBUF):
            g = i * NBUF + b

            @pl.when(g + NBUF < ng)
            def _():
                wait_out(b)
                start_in(g + NBUF, b)

        return 0

    lax.fori_loop(0, ng // NBUF, rows, 0)
    for b in range(ng - (ng // NBUF) * NBUF):
        wait_in(b)
        start_out((ng // NBUF) * NBUF + b, b)
    for b in range(NBUF):
        wait_out(b)


def kernel(hidden_states, probs, routing_map):
    maskf = routing_map.T.astype(jnp.int32).reshape(-1)
    probsf = probs.T.reshape(-1)
    counts, selpad, prpad = _compact_kernel(maskf, probsf)
    permuted_tokens, tokens_per_expert, permuted_probs = _gather_kernel(
        hidden_states, counts, selpad, prpad)
    return permuted_tokens, tokens_per_expert, permuted_probs


# speedup vs baseline: 1.2254x; 1.2110x over previous
import functools

import jax
import jax.numpy as jnp
from jax import lax
from jax.experimental import pallas as pl
from jax.experimental.pallas import tpu as pltpu
from jax.experimental.pallas import tpu_sc as plsc

T = 16384
E = 8
D = 4096
TOPK = 2
FLAT = T * E
OUT = T * TOPK
NW = 32
CHUNK = FLAT // NW
TPW = T // NW
ROWS_PW = OUT // NW
L = 16
RG = 8
NBUF = 3

_MESH = dict(core_axis_name="c", subcore_axis_name="s")

_DNUMS = lax.GatherDimensionNumbers(
    offset_dims=(), collapsed_slice_dims=(0,), start_index_map=(0,))


def _wid():
    return lax.axis_index("s") * 2 + lax.axis_index("c")


def _dyn_gather(x, idx):
    return lax.gather(x, idx[:, None], dimension_numbers=_DNUMS,
                      slice_sizes=(1,),
                      mode=lax.GatherScatterMode.PROMISE_IN_BOUNDS)


def _iota():
    return lax.iota(jnp.int32, L)


def _shift_cumsum(x):
    iota = _iota()
    y = x
    for s in (1, 2, 4, 8):
        sh = _dyn_gather(y, jnp.maximum(iota - s, 0))
        y = y + jnp.where(iota >= s, sh, 0)
    return y


def _sum_scalar(v):
    return _shift_cumsum(v)[L - 1]


def _splat(x):
    return jnp.full((L,), x, jnp.int32)


@functools.partial(
    pl.kernel,
    out_type=(
        jax.ShapeDtypeStruct((NW * 8,), jnp.int32),
        jax.ShapeDtypeStruct((NW * 8,), jnp.int32),
        jax.ShapeDtypeStruct((FLAT,), jnp.float32),
    ),
    mesh=plsc.VectorSubcoreMesh(**_MESH),
    compiler_params=pltpu.CompilerParams(needs_layout_passes=False),
    scratch_types=[
        pltpu.VMEM((CHUNK,), jnp.int32),
        pltpu.VMEM((CHUNK,), jnp.float32),
        pltpu.VMEM((CHUNK,), jnp.float32),
        pltpu.VMEM((L,), jnp.int32),
        pltpu.VMEM((L,), jnp.int32),
    ],
)
def _compact_kernel(maskf_hbm, probsf_hbm, counts_hbm, counts3_hbm,
                    prpad_hbm, chunk_v, pchunk_v, pr_buf, cs_v, c3_v):
    w = _wid()
    iota = _iota()

    pltpu.sync_copy(maskf_hbm.at[pl.ds(w * CHUNK, CHUNK)], chunk_v)
    pltpu.sync_copy(probsf_hbm.at[pl.ds(w * CHUNK, CHUNK)], pchunk_v)

    def comp(g, off_vec):
        m = chunk_v[pl.ds(g * L, L)]
        mask = m == 1
        incl = _shift_cumsum(m)
        dst = jnp.maximum(off_vec + incl - 1, 0)
        plsc.store_scatter(pr_buf, [dst], pchunk_v[pl.ds(g * L, L)],
                           mask=mask)
        return off_vec + plsc.all_reduce_population_count(mask)

    off_vec = lax.fori_loop(0, CHUNK // L, comp, jnp.zeros((L,), jnp.int32),
                            unroll=4)
    cs_v[...] = off_vec
    pltpu.sync_copy(cs_v.at[pl.ds(0, 8)], counts_hbm.at[pl.ds(w * 8, 8)])
    pltpu.sync_copy(pr_buf, prpad_hbm.at[pl.ds(w * CHUNK, CHUNK)])

    c3 = jnp.zeros((L,), jnp.int32)
    for j in range(8):
        def sub(i, acc):
            return acc + chunk_v[pl.ds(j * TPW + i * L, L)]

        s = _sum_scalar(lax.fori_loop(0, TPW // L, sub,
                                      jnp.zeros((L,), jnp.int32), unroll=8))
        c3 = c3 + s * jnp.where(iota == j, 1, 0)
    c3_v[...] = c3
    pltpu.sync_copy(c3_v.at[pl.ds(0, 8)], counts3_hbm.at[pl.ds(w * 8, 8)])


@functools.partial(
    pl.kernel,
    out_type=(
        jax.ShapeDtypeStruct((OUT, D), jnp.float32),
        jax.ShapeDtypeStruct((E,), jnp.int32),
        jax.ShapeDtypeStruct((OUT,), jnp.float32),
    ),
    mesh=plsc.VectorSubcoreMesh(**_MESH),
    compiler_params=pltpu.CompilerParams(needs_layout_passes=False),
    scratch_types=[
        pltpu.VMEM((NW * 8,), jnp.int32),
        pltpu.VMEM((NW * 8,), jnp.int32),
        pltpu.VMEM((NW,), jnp.int32),
        pltpu.VMEM((NW,), jnp.int32),
        pltpu.VMEM((CHUNK,), jnp.int32),
        pltpu.VMEM((CHUNK,), jnp.float32),
        pltpu.VMEM((ROWS_PW,), jnp.float32),
        pltpu.VMEM((TPW // RG, RG), jnp.int32),
        pltpu.VMEM((TPW // RG, RG), jnp.int32),
        pltpu.VMEM((L,), jnp.int32),
        [pltpu.VMEM((RG, D), jnp.float32) for _ in range(NBUF)],
        [pltpu.SemaphoreType.DMA for _ in range(NBUF)],
        [pltpu.SemaphoreType.DMA for _ in range(NBUF)],
        [pltpu.SemaphoreType.DMA for _ in range(NBUF)],
    ],
)
def _gather_kernel(hidden_hbm, maskr_hbm, counts_hbm, counts3_hbm, prpad_hbm,
                   out_hbm, tpe_hbm, pprobs_hbm,
                   cnt_v, cnt3_v, pfx_v, ca_v, region_v, prow_v, pr_v,
                   dstA, dstB, tpe_v, rbufs, isems, asems, bsems):
    w = _wid()
    iota = _iota()

    pltpu.sync_copy(counts_hbm, cnt_v)
    pltpu.sync_copy(counts3_hbm, cnt3_v)
    c0 = plsc.load_gather(cnt_v, [iota * 8])
    c1 = plsc.load_gather(cnt_v, [(iota + L) * 8])

    @pl.when(w == 0)
    def _():
        acc = jnp.zeros((L,), jnp.int32)
        for e in range(E):
            src = c0 if e < 4 else c1
            s = _sum_scalar(jnp.where(iota // 4 == (e % 4), src, 0))
            acc = acc + s * jnp.where(iota == e, 1, 0)
        tpe_v[...] = acc
        pltpu.sync_copy(tpe_v.at[pl.ds(0, E)], tpe_hbm)

    q = w // 8
    j0 = w % 8
    colbase = jnp.zeros((L,), jnp.int32)
    run = jnp.int32(0)
    for e in range(E):
        src = c0 if e < 4 else c1
        lane0 = (e % 4) * 4
        tot = _sum_scalar(jnp.where(iota // 4 == (e % 4), src, 0))
        before = _sum_scalar(
            jnp.where((iota >= lane0) & (iota < lane0 + q), src, 0))
        sub = plsc.load_gather(cnt3_v, [_splat((4 * e + q) * 8)
                                        + jnp.minimum(iota, 7)])
        partial = _sum_scalar(jnp.where(iota < j0, sub, 0))
        base_e = run + before + partial
        colbase = colbase + base_e * jnp.where((iota == e) | (iota == e + 8),
                                               1, 0)
        run = run + tot

    pltpu.sync_copy(maskr_hbm.at[pl.ds(w * CHUNK, CHUNK)], region_v)

    def emit(g, runrank):
        m = region_v[pl.ds(g * L, L)]
        is_set = m == 1
        incl = _shift_cumsum(m)
        b7 = _dyn_gather(incl, _splat(7))
        kv = incl - m - jnp.where(iota >= 8, b7, 0)
        add_t1 = jnp.where(iota >= 8,
                           _dyn_gather(m, jnp.maximum(iota - 8, 0)), 0)
        dest = colbase + runrank + add_t1
        tloc = _splat(2 * g) + jnp.where(iota >= 8, 1, 0)
        plsc.store_scatter(dstA, [lax.shift_right_logical(tloc, 3),
                                  lax.bitwise_and(tloc, 7)],
                           dest, mask=is_set & (kv == 0))
        plsc.store_scatter(dstB, [lax.shift_right_logical(tloc, 3),
                                  lax.bitwise_and(tloc, 7)],
                           dest, mask=is_set & (kv == 1))
        lo = lax.bitwise_and(iota, 7)
        return runrank + _dyn_gather(m, lo) + _dyn_gather(m, lo + 8)

    lax.fori_loop(0, CHUNK // L, emit, jnp.zeros((L,), jnp.int32), unroll=4)

    p0 = _shift_cumsum(c0) - c0
    s0 = _sum_scalar(c0)
    p1 = _shift_cumsum(c1) - c1 + s0
    pfx_v[pl.ds(0, L)] = p0
    pfx_v[pl.ds(L, L)] = p1
    ca_v[pl.ds(0, L)] = c0
    ca_v[pl.ds(L, L)] = c1
    row0 = w * ROWS_PW

    def pull(u, _):
        pu = plsc.load_gather(pfx_v, [_splat(u)])[0]
        cu = plsc.load_gather(ca_v, [_splat(u)])[0]
        a = jnp.maximum(pu, row0)
        b = jnp.minimum(pu + cu, row0 + ROWS_PW)
        n = b - a

        @pl.when(n > 0)
        def _():
            pltpu.sync_copy(prpad_hbm.at[pl.ds(u * CHUNK, CHUNK)], prow_v)

            def cp(g, _):
                off = g * L + iota
                valid = off < n
                src = jnp.minimum(a - pu + off, CHUNK - 1)
                dst = jnp.minimum(a - row0 + off, ROWS_PW - 1)
                plsc.store_scatter(pr_v, [dst],
                                   plsc.load_gather(prow_v, [src]),
                                   mask=valid)
                return 0

            lax.fori_loop(0, (n + L - 1) // L, cp, 0)

        return 0

    lax.fori_loop(0, NW, pull, 0)
    pltpu.sync_copy(pr_v, pprobs_hbm.at[pl.ds(row0, ROWS_PW)])

    ng = TPW // RG
    tok0 = w * TPW

    def start_in(g, b):
        pltpu.async_copy(hidden_hbm.at[pl.ds(tok0 + g * RG, RG)],
                         rbufs[b], isems[b])

    def wait_in(b):
        pltpu.make_async_copy(hidden_hbm.at[pl.ds(tok0, RG)],
                              rbufs[b], isems[b]).wait()

    def start_out(g, b):
        pltpu.async_copy(rbufs[b], out_hbm.at[dstA.at[g]], asems[b])
        pltpu.async_copy(rbufs[b], out_hbm.at[dstB.at[g]], bsems[b])

    def wait_out(b):
        pltpu.make_async_copy(rbufs[b], out_hbm.at[dstA.at[0]],
                              asems[b]).wait()
        pltpu.make_async_copy(rbufs[b], out_hbm.at[dstB.at[0]],
                              bsems[b]).wait()

    for b in range(NBUF):
        start_in(b, b)

    def rows(i, _):
        for b in range(NBUF):
            wait_in(b)
            start_out(i * NBUF + b, b)
        for b in range(NBUF):
            g = i * NBUF + b

            @pl.when(g + NBUF < ng)
            def _():
                wait_out(b)
                start_in(g + NBUF, b)

        return 0

    lax.fori_loop(0, ng // NBUF, rows, 0)
    for b in range(ng - (ng // NBUF) * NBUF):
        wait_in(b)
        start_out((ng // NBUF) * NBUF + b, b)
    for b in range(NBUF):
        wait_out(b)


def kernel(hidden_states, probs, routing_map):
    maskf = routing_map.T.astype(jnp.int32).reshape(-1)
    maskr = routing_map.astype(jnp.int32).reshape(-1)
    probsf = probs.T.reshape(-1)
    counts, counts3, prpad = _compact_kernel(maskf, probsf)
    permuted_tokens, tokens_per_expert, permuted_probs = _gather_kernel(
        hidden_states, maskr, counts, counts3, prpad)
    return permuted_tokens, tokens_per_expert, permuted_probs


# speedup vs baseline: 1.2260x; 1.0005x over previous
import functools

import jax
import jax.numpy as jnp
from jax import lax
from jax.experimental import pallas as pl
from jax.experimental.pallas import tpu as pltpu
from jax.experimental.pallas import tpu_sc as plsc

T = 16384
E = 8
D = 4096
TOPK = 2
FLAT = T * E
OUT = T * TOPK
NW = 32
CHUNK = FLAT // NW
TPW = T // NW
ROWS_PW = OUT // NW
L = 16
RG = 8
NBUF = 3

_MESH = dict(core_axis_name="c", subcore_axis_name="s")

_DNUMS = lax.GatherDimensionNumbers(
    offset_dims=(), collapsed_slice_dims=(0,), start_index_map=(0,))


def _wid():
    return lax.axis_index("s") * 2 + lax.axis_index("c")


def _dyn_gather(x, idx):
    return lax.gather(x, idx[:, None], dimension_numbers=_DNUMS,
                      slice_sizes=(1,),
                      mode=lax.GatherScatterMode.PROMISE_IN_BOUNDS)


def _iota():
    return lax.iota(jnp.int32, L)


def _shift_cumsum(x):
    iota = _iota()
    y = x
    for s in (1, 2, 4, 8):
        sh = _dyn_gather(y, jnp.maximum(iota - s, 0))
        y = y + jnp.where(iota >= s, sh, 0)
    return y


def _sum_scalar(v):
    return _shift_cumsum(v)[L - 1]


def _splat(x):
    return jnp.full((L,), x, jnp.int32)


@functools.partial(
    pl.kernel,
    out_type=(
        jax.ShapeDtypeStruct((NW * 8,), jnp.int32),
        jax.ShapeDtypeStruct((NW * 8,), jnp.int32),
        jax.ShapeDtypeStruct((FLAT,), jnp.float32),
    ),
    mesh=plsc.VectorSubcoreMesh(**_MESH),
    compiler_params=pltpu.CompilerParams(needs_layout_passes=False),
    scratch_types=[
        pltpu.VMEM((CHUNK,), jnp.int32),
        pltpu.VMEM((CHUNK,), jnp.float32),
        pltpu.VMEM((CHUNK,), jnp.float32),
        pltpu.VMEM((L,), jnp.int32),
        pltpu.VMEM((L,), jnp.int32),
    ],
)
def _compact_kernel(maskf_hbm, probsf_hbm, counts_hbm, counts3_hbm,
                    prpad_hbm, chunk_v, pchunk_v, pr_buf, cs_v, c3_v):
    w = _wid()
    iota = _iota()

    pltpu.sync_copy(maskf_hbm.at[pl.ds(w * CHUNK, CHUNK)], chunk_v)
    pltpu.sync_copy(probsf_hbm.at[pl.ds(w * CHUNK, CHUNK)], pchunk_v)

    def comp(g, off_vec):
        m = chunk_v[pl.ds(g * L, L)]
        mask = m == 1
        incl = _shift_cumsum(m)
        dst = jnp.maximum(off_vec + incl - 1, 0)
        plsc.store_scatter(pr_buf, [dst], pchunk_v[pl.ds(g * L, L)],
                           mask=mask)
        return off_vec + plsc.all_reduce_population_count(mask)

    off_vec = lax.fori_loop(0, CHUNK // L, comp, jnp.zeros((L,), jnp.int32),
                            unroll=4)
    cs_v[...] = off_vec
    pltpu.sync_copy(cs_v.at[pl.ds(0, 8)], counts_hbm.at[pl.ds(w * 8, 8)])
    pltpu.sync_copy(pr_buf, prpad_hbm.at[pl.ds(w * CHUNK, CHUNK)])

    c3 = jnp.zeros((L,), jnp.int32)
    for j in range(8):
        def sub(i, acc):
            return acc + chunk_v[pl.ds(j * TPW + i * L, L)]

        s = _sum_scalar(lax.fori_loop(0, TPW // L, sub,
                                      jnp.zeros((L,), jnp.int32), unroll=8))
        c3 = c3 + s * jnp.where(iota == j, 1, 0)
    c3_v[...] = c3
    pltpu.sync_copy(c3_v.at[pl.ds(0, 8)], counts3_hbm.at[pl.ds(w * 8, 8)])


@functools.partial(
    pl.kernel,
    out_type=(
        jax.ShapeDtypeStruct((OUT, D), jnp.float32),
        jax.ShapeDtypeStruct((E,), jnp.int32),
        jax.ShapeDtypeStruct((OUT,), jnp.float32),
    ),
    mesh=plsc.VectorSubcoreMesh(**_MESH),
    compiler_params=pltpu.CompilerParams(needs_layout_passes=False),
    scratch_types=[
        pltpu.VMEM((NW * 8,), jnp.int32),
        pltpu.VMEM((NW * 8,), jnp.int32),
        pltpu.VMEM((NW,), jnp.int32),
        pltpu.VMEM((NW,), jnp.int32),
        pltpu.VMEM((CHUNK,), jnp.int32),
        pltpu.VMEM((CHUNK,), jnp.float32),
        pltpu.VMEM((ROWS_PW,), jnp.float32),
        pltpu.VMEM((TPW // RG, RG), jnp.int32),
        pltpu.VMEM((TPW // RG, RG), jnp.int32),
        pltpu.VMEM((L,), jnp.int32),
        [pltpu.VMEM((RG, D), jnp.float32) for _ in range(NBUF)],
        [pltpu.SemaphoreType.DMA for _ in range(NBUF)],
        [pltpu.SemaphoreType.DMA for _ in range(NBUF)],
        [pltpu.SemaphoreType.DMA for _ in range(NBUF)],
    ],
)
def _gather_kernel(hidden_hbm, maskr_hbm, counts_hbm, counts3_hbm, prpad_hbm,
                   out_hbm, tpe_hbm, pprobs_hbm,
                   cnt_v, cnt3_v, pfx_v, ca_v, region_v, prow_v, pr_v,
                   dstA, dstB, tpe_v, rbufs, isems, asems, bsems):
    w = _wid()
    iota = _iota()
    tok0 = w * TPW

    for b in range(NBUF):
        pltpu.async_copy(hidden_hbm.at[pl.ds(tok0 + b * RG, RG)],
                         rbufs[b], isems[b])

    pltpu.sync_copy(counts_hbm, cnt_v)
    pltpu.sync_copy(counts3_hbm, cnt3_v)
    c0 = plsc.load_gather(cnt_v, [iota * 8])
    c1 = plsc.load_gather(cnt_v, [(iota + L) * 8])

    @pl.when(w == 0)
    def _():
        acc = jnp.zeros((L,), jnp.int32)
        for e in range(E):
            src = c0 if e < 4 else c1
            s = _sum_scalar(jnp.where(iota // 4 == (e % 4), src, 0))
            acc = acc + s * jnp.where(iota == e, 1, 0)
        tpe_v[...] = acc
        pltpu.sync_copy(tpe_v.at[pl.ds(0, E)], tpe_hbm)

    q = w // 8
    j0 = w % 8
    colbase = jnp.zeros((L,), jnp.int32)
    run = jnp.int32(0)
    for e in range(E):
        src = c0 if e < 4 else c1
        lane0 = (e % 4) * 4
        tot = _sum_scalar(jnp.where(iota // 4 == (e % 4), src, 0))
        before = _sum_scalar(
            jnp.where((iota >= lane0) & (iota < lane0 + q), src, 0))
        sub = plsc.load_gather(cnt3_v, [_splat((4 * e + q) * 8)
                                        + jnp.minimum(iota, 7)])
        partial = _sum_scalar(jnp.where(iota < j0, sub, 0))
        base_e = run + before + partial
        colbase = colbase + base_e * jnp.where((iota == e) | (iota == e + 8),
                                               1, 0)
        run = run + tot

    pltpu.sync_copy(maskr_hbm.at[pl.ds(w * CHUNK, CHUNK)], region_v)

    def emit(g, runrank):
        m = region_v[pl.ds(g * L, L)]
        is_set = m == 1
        incl = _shift_cumsum(m)
        b7 = _dyn_gather(incl, _splat(7))
        kv = incl - m - jnp.where(iota >= 8, b7, 0)
        add_t1 = jnp.where(iota >= 8,
                           _dyn_gather(m, jnp.maximum(iota - 8, 0)), 0)
        dest = colbase + runrank + add_t1
        tloc = _splat(2 * g) + jnp.where(iota >= 8, 1, 0)
        plsc.store_scatter(dstA, [lax.shift_right_logical(tloc, 3),
                                  lax.bitwise_and(tloc, 7)],
                           dest, mask=is_set & (kv == 0))
        plsc.store_scatter(dstB, [lax.shift_right_logical(tloc, 3),
                                  lax.bitwise_and(tloc, 7)],
                           dest, mask=is_set & (kv == 1))
        lo = lax.bitwise_and(iota, 7)
        return runrank + _dyn_gather(m, lo) + _dyn_gather(m, lo + 8)

    lax.fori_loop(0, CHUNK // L, emit, jnp.zeros((L,), jnp.int32), unroll=4)

    p0 = _shift_cumsum(c0) - c0
    s0 = _sum_scalar(c0)
    p1 = _shift_cumsum(c1) - c1 + s0
    pfx_v[pl.ds(0, L)] = p0
    pfx_v[pl.ds(L, L)] = p1
    ca_v[pl.ds(0, L)] = c0
    ca_v[pl.ds(L, L)] = c1
    row0 = w * ROWS_PW

    def pull(u, _):
        pu = plsc.load_gather(pfx_v, [_splat(u)])[0]
        cu = plsc.load_gather(ca_v, [_splat(u)])[0]
        a = jnp.maximum(pu, row0)
        b = jnp.minimum(pu + cu, row0 + ROWS_PW)
        n = b - a

        @pl.when(n > 0)
        def _():
            pltpu.sync_copy(prpad_hbm.at[pl.ds(u * CHUNK, CHUNK)], prow_v)

            def cp(g, _):
                off = g * L + iota
                valid = off < n
                src = jnp.minimum(a - pu + off, CHUNK - 1)
                dst = jnp.minimum(a - row0 + off, ROWS_PW - 1)
                plsc.store_scatter(pr_v, [dst],
                                   plsc.load_gather(prow_v, [src]),
                                   mask=valid)
                return 0

            lax.fori_loop(0, (n + L - 1) // L, cp, 0)

        return 0

    lax.fori_loop(0, NW, pull, 0)
    pltpu.sync_copy(pr_v, pprobs_hbm.at[pl.ds(row0, ROWS_PW)])

    ng = TPW // RG

    def start_in(g, b):
        pltpu.async_copy(hidden_hbm.at[pl.ds(tok0 + g * RG, RG)],
                         rbufs[b], isems[b])

    def wait_in(b):
        pltpu.make_async_copy(hidden_hbm.at[pl.ds(tok0, RG)],
                              rbufs[b], isems[b]).wait()

    def start_out(g, b):
        pltpu.async_copy(rbufs[b], out_hbm.at[dstA.at[g]], asems[b])
        pltpu.async_copy(rbufs[b], out_hbm.at[dstB.at[g]], bsems[b])

    def wait_out(b):
        pltpu.make_async_copy(rbufs[b], out_hbm.at[dstA.at[0]],
                              asems[b]).wait()
        pltpu.make_async_copy(rbufs[b], out_hbm.at[dstB.at[0]],
                              bsems[b]).wait()

    def rows(i, _):
        for b in range(NBUF):
            wait_in(b)
            start_out(i * NBUF + b, b)
        for b in range(NBUF):
            g = i * NBUF + b

            @pl.when(g + NBUF < ng)
            def _():
                wait_out(b)
                start_in(g + NBUF, b)

        return 0

    lax.fori_loop(0, ng // NBUF, rows, 0)
    for b in range(ng - (ng // NBUF) * NBUF):
        wait_in(b)
        start_out((ng // NBUF) * NBUF + b, b)
    for b in range(NBUF):
        wait_out(b)


def kernel(hidden_states, probs, routing_map):
    maskf = routing_map.T.astype(jnp.int32).reshape(-1)
    maskr = routing_map.astype(jnp.int32).reshape(-1)
    probsf = probs.T.reshape(-1)
    counts, counts3, prpad = _compact_kernel(maskf, probsf)
    permuted_tokens, tokens_per_expert, permuted_probs = _gather_kernel(
        hidden_states, maskr, counts, counts3, prpad)
    return permuted_tokens, tokens_per_expert, permuted_probs
